# baseline (device time: 85343 ns/iter reference)
import jax
import jax.numpy as jnp
from jax import lax
from jax.experimental import pallas as pl
from jax.experimental.pallas import tpu as pltpu

T = 2048
HALF = T // 2
N_SUB = 16
SUB = HALF // N_SUB
G_SUB = 16
G_ROWS = T // G_SUB


def _gather_exchange_select(safe_ids, mask, E):
    d = E.shape[1]

    def body(ids_ref, mask_ref, e_ref, out_ref, partial_ref, comm_ref,
             gather_sems, ysend_sems, asend_sems, bsend_sems,
             arecv_sems, brecv_sems):
        my_x = lax.axis_index("x")
        my_y = lax.axis_index("y")
        my_z = lax.axis_index("z")
        partner = (my_x, 1 - my_y, my_z)
        right = (my_x, my_y, my_z + 1)
        left = (my_x, my_y, my_z - 1)
        is_z0 = my_z == 0
        is_z3 = my_z == 3

        goff = jnp.where(is_z3, HALF, 0)

        def issue_gather(g):
            def issue(i4, _):
                for u in range(4):
                    r = lax.rem(goff + g * G_ROWS + i4 * 4 + u, T)
                    pltpu.make_async_copy(
                        e_ref.at[pl.ds(ids_ref[r], 1), :],
                        partial_ref.at[pl.ds(r, 1), :],
                        gather_sems.at[g],
                    ).start()
                return _

            lax.fori_loop(0, G_ROWS // 4, issue, None)

        def drain_gather(g):
            pltpu.make_async_copy(
                e_ref.at[pl.ds(0, G_ROWS), :],
                partial_ref.at[pl.ds(0, G_ROWS), :],
                gather_sems.at[g],
            ).wait()

        issue_gather(0)
        issue_gather(1)

        barrier_sem = pltpu.get_barrier_semaphore()

        def sig(dev):
            pl.semaphore_signal(
                barrier_sem, inc=1, device_id=dev,
                device_id_type=pl.DeviceIdType.MESH,
            )

        @pl.when(is_z0)
        def _():
            sig(partner)
            sig(right)

        @pl.when(jnp.logical_and(my_z > 0, my_z < 3))
        def _():
            sig(left)
            sig(right)

        @pl.when(is_z3)
        def _():
            sig(left)
            sig(partner)

        pl.semaphore_wait(barrier_sem, 2)

        def y_send_h1(s):
            return pltpu.make_async_remote_copy(
                src_ref=partial_ref.at[pl.ds(s * SUB, SUB), :],
                dst_ref=comm_ref.at[pl.ds(s * SUB, SUB), :],
                send_sem=ysend_sems.at[s],
                recv_sem=arecv_sems.at[s],
                device_id=partner,
                device_id_type=pl.DeviceIdType.MESH,
            )

        def y_send_h2(s):
            return pltpu.make_async_remote_copy(
                src_ref=partial_ref.at[pl.ds(HALF + s * SUB, SUB), :],
                dst_ref=comm_ref.at[pl.ds(HALF + s * SUB, SUB), :],
                send_sem=ysend_sems.at[s],
                recv_sem=brecv_sems.at[s],
                device_id=partner,
                device_id_type=pl.DeviceIdType.MESH,
            )

        def fwd_a(s):
            return pltpu.make_async_remote_copy(
                src_ref=comm_ref.at[pl.ds(s * SUB, SUB), :],
                dst_ref=comm_ref.at[pl.ds(s * SUB, SUB), :],
                send_sem=asend_sems.at[s],
                recv_sem=arecv_sems.at[s],
                device_id=right,
                device_id_type=pl.DeviceIdType.MESH,
            )

        def fwd_b(s):
            return pltpu.make_async_remote_copy(
                src_ref=comm_ref.at[pl.ds(HALF + s * SUB, SUB), :],
                dst_ref=comm_ref.at[pl.ds(HALF + s * SUB, SUB), :],
                send_sem=bsend_sems.at[s],
                recv_sem=brecv_sems.at[s],
                device_id=left,
                device_id_type=pl.DeviceIdType.MESH,
            )

        def role_step(k, lag_a, lag_b, ysend, fa_start, fb_start):
            if ysend is not None and k < N_SUB:
                ysend(k).start()
            sa = k - lag_a
            if 0 <= sa < N_SUB:
                fwd_a(sa).wait_recv()
                if fa_start:
                    fwd_a(sa).start()
            sb = k - lag_b
            if 0 <= sb < N_SUB:
                fwd_b(sb).wait_recv()
                if fb_start:
                    fwd_b(sb).start()

        STEPS = N_SUB + 7
        for k in range(STEPS):
            if k + 2 < G_SUB:
                issue_gather(k + 2)
            if k % 2 == 0 and k // 2 < G_SUB // 2:
                drain_gather(k // 2)

            @pl.when(is_z0)
            def _(k=k):
                role_step(k, 2, 7, y_send_h1, True, False)

            @pl.when(my_z == 1)
            def _(k=k):
                role_step(k, 4, 5, None, True, True)

            @pl.when(my_z == 2)
            def _(k=k):
                role_step(k, 5, 4, None, True, True)

            @pl.when(is_z3)
            def _(k=k):
                role_step(k, 7, 2, y_send_h2, False, True)

        for g in range(G_SUB // 2, G_SUB):
            drain_gather(g)

        out_ref[...] = jnp.where(
            mask_ref[...] > 0.5, partial_ref[...], comm_ref[...]
        )

        @pl.when(jnp.logical_or(is_z0, is_z3))
        def _():
            for s in range(N_SUB):
                y_send_h1(s).wait_send()

        @pl.when(my_z < 3)
        def _():
            for s in range(N_SUB):
                fwd_a(s).wait_send()

        @pl.when(my_z > 0)
        def _():
            for s in range(N_SUB):
                fwd_b(s).wait_send()

    return pl.pallas_call(
        body,
        out_shape=jax.ShapeDtypeStruct((T, d), jnp.float32),
        in_specs=[
            pl.BlockSpec(memory_space=pltpu.SMEM),
            pl.BlockSpec(memory_space=pltpu.VMEM),
            pl.BlockSpec(memory_space=pl.ANY),
        ],
        out_specs=pl.BlockSpec(memory_space=pltpu.VMEM),
        scratch_shapes=[
            pltpu.VMEM((T, d), jnp.float32),
            pltpu.VMEM((T, d), jnp.float32),
            pltpu.SemaphoreType.DMA((G_SUB,)),
            pltpu.SemaphoreType.DMA((N_SUB,)),
            pltpu.SemaphoreType.DMA((N_SUB,)),
            pltpu.SemaphoreType.DMA((N_SUB,)),
            pltpu.SemaphoreType.DMA((N_SUB,)),
            pltpu.SemaphoreType.DMA((N_SUB,)),
        ],
        compiler_params=pltpu.CompilerParams(collective_id=0),
    )(safe_ids, mask, E)


def kernel(ids, E):
    my_y = lax.axis_index("y")
    v_shard = E.shape[0]
    local = ids - my_y * v_shard
    in_range = (local >= 0) & (local < v_shard)
    safe = jnp.where(in_range, local, 0).astype(jnp.int32)
    mask = in_range[:, None].astype(jnp.float32)
    return _gather_exchange_select(safe, mask, E)


# device time: 77109 ns/iter; 1.1068x vs baseline; 1.1068x over previous
import jax
import jax.numpy as jnp
from jax import lax
from jax.experimental import pallas as pl
from jax.experimental.pallas import tpu as pltpu

T = 2048
HALF = T // 2
N_SUB = 16
SUB = HALF // N_SUB
G_SUB = 16
G_ROWS = T // G_SUB


def _gather_exchange_select(safe_ids, mask, E):
    d = E.shape[1]

    def body(ids_ref, mask_ref, e_ref, out_ref, partial_ref, comm_ref,
             gather_sems, ysend_sems, asend_sems, bsend_sems,
             arecv_sems, brecv_sems):
        my_x = lax.axis_index("x")
        my_y = lax.axis_index("y")
        my_z = lax.axis_index("z")
        partner = (my_x, 1 - my_y, my_z)
        right = (my_x, my_y, my_z + 1)
        left = (my_x, my_y, my_z - 1)
        is_z0 = my_z == 0
        is_z3 = my_z == 3

        goff = jnp.where(is_z3, HALF, 0)

        def issue_gather(g):
            def issue(i4, _):
                for u in range(4):
                    r = lax.rem(goff + g * G_ROWS + i4 * 4 + u, T)
                    pltpu.make_async_copy(
                        e_ref.at[pl.ds(ids_ref[r], 1), :],
                        partial_ref.at[pl.ds(r, 1), :],
                        gather_sems.at[g],
                    ).start()
                return _

            lax.fori_loop(0, G_ROWS // 4, issue, None)

        def drain_gather(g):
            pltpu.make_async_copy(
                e_ref.at[pl.ds(0, G_ROWS), :],
                partial_ref.at[pl.ds(0, G_ROWS), :],
                gather_sems.at[g],
            ).wait()

        issue_gather(0)
        issue_gather(1)

        barrier_sem = pltpu.get_barrier_semaphore()

        def sig(dev):
            pl.semaphore_signal(
                barrier_sem, inc=1, device_id=dev,
                device_id_type=pl.DeviceIdType.MESH,
            )

        @pl.when(is_z0)
        def _():
            sig(partner)
            sig(right)

        @pl.when(jnp.logical_and(my_z > 0, my_z < 3))
        def _():
            sig(left)
            sig(right)

        @pl.when(is_z3)
        def _():
            sig(left)
            sig(partner)

        pl.semaphore_wait(barrier_sem, 2)

        def y_send_h1(s):
            return pltpu.make_async_remote_copy(
                src_ref=partial_ref.at[pl.ds(s * SUB, SUB), :],
                dst_ref=comm_ref.at[pl.ds(s * SUB, SUB), :],
                send_sem=ysend_sems.at[s],
                recv_sem=arecv_sems.at[s],
                device_id=partner,
                device_id_type=pl.DeviceIdType.MESH,
            )

        def y_send_h2(s):
            return pltpu.make_async_remote_copy(
                src_ref=partial_ref.at[pl.ds(HALF + s * SUB, SUB), :],
                dst_ref=comm_ref.at[pl.ds(HALF + s * SUB, SUB), :],
                send_sem=ysend_sems.at[s],
                recv_sem=brecv_sems.at[s],
                device_id=partner,
                device_id_type=pl.DeviceIdType.MESH,
            )

        def fwd_a(s):
            return pltpu.make_async_remote_copy(
                src_ref=comm_ref.at[pl.ds(s * SUB, SUB), :],
                dst_ref=comm_ref.at[pl.ds(s * SUB, SUB), :],
                send_sem=asend_sems.at[s],
                recv_sem=arecv_sems.at[s],
                device_id=right,
                device_id_type=pl.DeviceIdType.MESH,
            )

        def fwd_b(s):
            return pltpu.make_async_remote_copy(
                src_ref=comm_ref.at[pl.ds(HALF + s * SUB, SUB), :],
                dst_ref=comm_ref.at[pl.ds(HALF + s * SUB, SUB), :],
                send_sem=bsend_sems.at[s],
                recv_sem=brecv_sems.at[s],
                device_id=left,
                device_id_type=pl.DeviceIdType.MESH,
            )

        def role_step(k, lag_a, lag_b, ysend, fa_start, fb_start):
            if ysend is not None and k < N_SUB:
                ysend(k).start()
            sa = k - lag_a
            if 0 <= sa < N_SUB:
                fwd_a(sa).wait_recv()
                if fa_start:
                    fwd_a(sa).start()
            sb = k - lag_b
            if 0 <= sb < N_SUB:
                fwd_b(sb).wait_recv()
                if fb_start:
                    fwd_b(sb).start()

        STEPS = N_SUB + 8
        for k in range(STEPS):
            if k + 2 < G_SUB:
                issue_gather(k + 2)
            if k % 2 == 0 and k // 2 < G_SUB // 2:
                drain_gather(k // 2)

            @pl.when(is_z0)
            def _(k=k):
                role_step(k, 2, 8, y_send_h1, True, False)

            @pl.when(my_z == 1)
            def _(k=k):
                role_step(k, 4, 6, None, True, True)

            @pl.when(my_z == 2)
            def _(k=k):
                role_step(k, 6, 4, None, True, True)

            @pl.when(is_z3)
            def _(k=k):
                role_step(k, 8, 2, y_send_h2, False, True)

        for g in range(G_SUB // 2, G_SUB):
            drain_gather(g)

        out_ref[...] = jnp.where(
            mask_ref[...] > 0.5, partial_ref[...], comm_ref[...]
        )

        @pl.when(jnp.logical_or(is_z0, is_z3))
        def _():
            for s in range(N_SUB):
                y_send_h1(s).wait_send()

        @pl.when(my_z < 3)
        def _():
            for s in range(N_SUB):
                fwd_a(s).wait_send()

        @pl.when(my_z > 0)
        def _():
            for s in range(N_SUB):
                fwd_b(s).wait_send()

    return pl.pallas_call(
        body,
        out_shape=jax.ShapeDtypeStruct((T, d), jnp.float32),
        in_specs=[
            pl.BlockSpec(memory_space=pltpu.SMEM),
            pl.BlockSpec(memory_space=pltpu.VMEM),
            pl.BlockSpec(memory_space=pl.ANY),
        ],
        out_specs=pl.BlockSpec(memory_space=pltpu.VMEM),
        scratch_shapes=[
            pltpu.VMEM((T, d), jnp.float32),
            pltpu.VMEM((T, d), jnp.float32),
            pltpu.SemaphoreType.DMA((G_SUB,)),
            pltpu.SemaphoreType.DMA((N_SUB,)),
            pltpu.SemaphoreType.DMA((N_SUB,)),
            pltpu.SemaphoreType.DMA((N_SUB,)),
            pltpu.SemaphoreType.DMA((N_SUB,)),
            pltpu.SemaphoreType.DMA((N_SUB,)),
        ],
        compiler_params=pltpu.CompilerParams(collective_id=0),
    )(safe_ids, mask, E)


def kernel(ids, E):
    my_y = lax.axis_index("y")
    v_shard = E.shape[0]
    local = ids - my_y * v_shard
    in_range = (local >= 0) & (local < v_shard)
    safe = jnp.where(in_range, local, 0).astype(jnp.int32)
    mask = in_range[:, None].astype(jnp.float32)
    return _gather_exchange_select(safe, mask, E)
